# Initial kernel scaffold; baseline (speedup 1.0000x reference)
import jax
import jax.numpy as jnp
from jax.experimental import pallas as pl


def kernel(x, edge_index, edge_attr, edge_couples, W_neigh0, b_neigh0, W_edge0, b_edge0, W_neigh1, b_neigh1, W_edge1, b_edge1):
    raise NotImplementedError("write your pallas kernel here")


# speedup vs baseline: 5.8968x; 5.8968x over previous
import jax
import jax.numpy as jnp
from jax import lax
from jax.experimental import pallas as pl
from jax.experimental.pallas import tpu as pltpu
from jax.experimental.pallas import tpu_sc as plsc

N_NODES = 100000
N_EDGES = 3200000
EDIM = 16
NDIM = 128
NB = 100000

NC = 2
NS = 16
NW = NC * NS

_IW = 125
_ROWS = N_EDGES // _IW
_RPT16 = _ROWS // NS
_BATCH = 8
_NBATCH = _RPT16 // _BATCH
_RPT = 6256
_RPT_LAST = N_NODES - 15 * _RPT


def _seg_body(dst2, attr, zsum, ones_h, out_tbl,
              tbl, idx2, attr_v, ones_v):
    c = lax.axis_index("c")
    s = lax.axis_index("s")
    @pl.when(s < NS - 1)
    def _():
        pltpu.sync_copy(zsum, tbl.at[pl.ds(s * _RPT, _RPT), :])

    @pl.when(s == NS - 1)
    def _():
        pltpu.sync_copy(zsum.at[pl.ds(0, _RPT_LAST), :],
                        tbl.at[pl.ds(s * _RPT, _RPT_LAST), :])

    pltpu.sync_copy(ones_h, ones_v)
    plsc.subcore_barrier()

    row0 = s * _RPT16

    @pl.when(c == 0)
    def _():
        def batch(b, carry):
            r0 = row0 + b * _BATCH
            e0 = r0 * _IW
            pltpu.sync_copy(dst2.at[pl.ds(r0, _BATCH), :], idx2)
            pltpu.sync_copy(attr.at[pl.ds(e0, _BATCH * _IW), :], attr_v)
            for j in range(_BATCH):
                pltpu.sync_copy(attr_v.at[pl.ds(j * _IW, _IW), :],
                                tbl.at[idx2.at[j]], add=True)
            return carry

        lax.fori_loop(0, _NBATCH, batch, 0)

    @pl.when(c == 1)
    def _():
        def batch(b, carry):
            r0 = row0 + b * _BATCH
            pltpu.sync_copy(dst2.at[pl.ds(r0, _BATCH), :], idx2)
            for j in range(_BATCH):
                pltpu.sync_copy(ones_v, tbl.at[idx2.at[j]], add=True)
            return carry

        lax.fori_loop(0, _NBATCH, batch, 0)

    plsc.subcore_barrier()

    @pl.when(s < NS - 1)
    def _():
        pltpu.sync_copy(tbl.at[pl.ds(s * _RPT, _RPT), :],
                        out_tbl.at[c, pl.ds(s * _RPT, _RPT), :])

    @pl.when(s == NS - 1)
    def _():
        pltpu.sync_copy(tbl.at[pl.ds(s * _RPT, _RPT_LAST), :],
                        out_tbl.at[c, pl.ds(s * _RPT, _RPT_LAST), :])


_sc_mesh = plsc.VectorSubcoreMesh(core_axis_name="c", subcore_axis_name="s",
                                  num_cores=NC, num_subcores=NS)
_sc_params = pltpu.CompilerParams(use_tc_tiling_on_sc=False)

_seg = pl.kernel(
    _seg_body,
    out_type=jax.ShapeDtypeStruct((NC, N_NODES, EDIM), jnp.float32),
    mesh=_sc_mesh,
    scratch_types=[
        pltpu.VMEM_SHARED((N_NODES, EDIM), jnp.float32),
        pltpu.VMEM((_BATCH, _IW), jnp.int32),
        pltpu.VMEM((_BATCH * _IW, EDIM), jnp.float32),
        pltpu.VMEM((_IW, EDIM), jnp.float32),
    ],
    compiler_params=_sc_params,
)

_BK = 1000


def _dense_body(tbl_ref, wn_ref, bn_ref, wp_ref, wq_ref, be_ref,
                p_ref, q_ref):
    s = tbl_ref[0]
    cnt = tbl_ref[1, :, 0:1]
    mean = s / jnp.maximum(cnt, 1.0)
    h = jnp.dot(mean, wn_ref[...], preferred_element_type=jnp.float32)
    h = jnp.maximum(h + bn_ref[...], 0.0)
    p_ref[...] = jnp.dot(h, wp_ref[...],
                         preferred_element_type=jnp.float32) + be_ref[...]
    q_ref[...] = jnp.dot(h, wq_ref[...], preferred_element_type=jnp.float32)


_dense = pl.pallas_call(
    _dense_body,
    grid=(N_NODES // _BK,),
    in_specs=[
        pl.BlockSpec((NC, _BK, EDIM), lambda i: (0, i, 0)),
        pl.BlockSpec((EDIM, NDIM), lambda i: (0, 0)),
        pl.BlockSpec((1, NDIM), lambda i: (0, 0)),
        pl.BlockSpec((NDIM, NDIM), lambda i: (0, 0)),
        pl.BlockSpec((NDIM, NDIM), lambda i: (0, 0)),
        pl.BlockSpec((1, NDIM), lambda i: (0, 0)),
    ],
    out_specs=[pl.BlockSpec((_BK, NDIM), lambda i: (i, 0)),
               pl.BlockSpec((_BK, NDIM), lambda i: (i, 0))],
    out_shape=[jax.ShapeDtypeStruct((N_NODES, NDIM), jnp.float32),
               jax.ShapeDtypeStruct((N_NODES, NDIM), jnp.float32)],
)

_NJ = 25
_PPW = _NJ * 128
_NBP = NW * _PPW


def _pair_body(pt, qt, srcp, dstp, out, isrc, idst, gp, gq, ob, sem1, sem2):
    c = lax.axis_index("c")
    s = lax.axis_index("s")
    w = c * NS + s
    pltpu.sync_copy(srcp.at[w], isrc)
    pltpu.sync_copy(dstp.at[w], idst)
    base = w * _PPW

    def issue(j, carry):
        cp1 = pltpu.async_copy(pt.at[isrc.at[j]], gp, sem1)
        cp2 = pltpu.async_copy(qt.at[idst.at[j]], gq, sem2)
        cp1.wait()
        cp2.wait()

        def addrow(i, c2):
            for k in range(NDIM // 16):
                ob[i, pl.ds(k * 16, 16)] = (gp[i, pl.ds(k * 16, 16)] +
                                            gq[i, pl.ds(k * 16, 16)])
            return c2

        lax.fori_loop(0, 128, addrow, 0)
        pltpu.sync_copy(ob, out.at[pl.ds(base + j * 128, 128), :])
        return carry

    lax.fori_loop(0, _NJ, issue, 0)


_pair = pl.kernel(
    _pair_body,
    out_type=jax.ShapeDtypeStruct((_NBP, NDIM), jnp.float32),
    mesh=_sc_mesh,
    scratch_types=[
        pltpu.VMEM((_NJ, 128), jnp.int32),
        pltpu.VMEM((_NJ, 128), jnp.int32),
        pltpu.VMEM((128, NDIM), jnp.float32),
        pltpu.VMEM((128, NDIM), jnp.float32),
        pltpu.VMEM((128, NDIM), jnp.float32),
        pltpu.SemaphoreType.DMA,
        pltpu.SemaphoreType.DMA,
    ],
    compiler_params=_sc_params,
)


def kernel(x, edge_index, edge_attr, edge_couples,
           W_neigh0, b_neigh0, W_edge0, b_edge0,
           W_neigh1, b_neigh1, W_edge1, b_edge1):
    del x
    dst = edge_index[1]
    dst2 = dst.reshape(_ROWS, _IW)
    zsum = jnp.zeros((_RPT, EDIM), jnp.float32)
    ones_h = jnp.ones((_IW, EDIM), jnp.float32)
    out_tbl = _seg(dst2, edge_attr, zsum, ones_h)

    wn = W_neigh1.T
    bn = b_neigh1.reshape(1, NDIM)
    wp = W_edge1[:, :NDIM].T
    wq = W_edge1[:, NDIM:].T
    be = b_edge1.reshape(1, NDIM)
    P, Q = _dense(out_tbl, wn, bn, wp, wq, be)

    def _pack_idx(v):
        v = jnp.pad(v, (0, _NBP - NB))
        return v.reshape(NW, _NJ, 128)

    e1 = _pair(P, Q, _pack_idx(edge_couples[:, 0]), _pack_idx(edge_couples[:, 1]))
    return e1[:NB]


# speedup vs baseline: 5.9242x; 1.0046x over previous
import jax
import jax.numpy as jnp
from jax import lax
from jax.experimental import pallas as pl
from jax.experimental.pallas import tpu as pltpu
from jax.experimental.pallas import tpu_sc as plsc

N_NODES = 100000
N_EDGES = 3200000
EDIM = 16
NDIM = 128
NB = 100000

NC = 2
NS = 16
NW = NC * NS

_IW = 125
_ROWS = N_EDGES // _IW
_RPT16 = _ROWS // NS
_BATCH = 8
_NBATCH = _RPT16 // _BATCH
_RPT = 6256
_RPT_LAST = N_NODES - 15 * _RPT


def _seg_body(dst2, attr, zsum, ones_h, out_tbl,
              tbl, idx2, attr_v, ones_v):
    c = lax.axis_index("c")
    s = lax.axis_index("s")
    @pl.when(s < NS - 1)
    def _():
        pltpu.sync_copy(zsum, tbl.at[pl.ds(s * _RPT, _RPT), :])

    @pl.when(s == NS - 1)
    def _():
        pltpu.sync_copy(zsum.at[pl.ds(0, _RPT_LAST), :],
                        tbl.at[pl.ds(s * _RPT, _RPT_LAST), :])

    pltpu.sync_copy(ones_h, ones_v)
    plsc.subcore_barrier()

    row0 = s * _RPT16

    @pl.when(c == 0)
    def _():
        def batch(b, carry):
            r0 = row0 + b * _BATCH
            e0 = r0 * _IW
            pltpu.sync_copy(dst2.at[pl.ds(r0, _BATCH), :], idx2)
            pltpu.sync_copy(attr.at[pl.ds(e0, _BATCH * _IW), :], attr_v)
            for j in range(_BATCH):
                pltpu.sync_copy(attr_v.at[pl.ds(j * _IW, _IW), :],
                                tbl.at[idx2.at[j]], add=True)
            return carry

        lax.fori_loop(0, _NBATCH, batch, 0)

    @pl.when(c == 1)
    def _():
        def batch(b, carry):
            r0 = row0 + b * _BATCH
            pltpu.sync_copy(dst2.at[pl.ds(r0, _BATCH), :], idx2)
            for j in range(_BATCH):
                pltpu.sync_copy(ones_v, tbl.at[idx2.at[j]], add=True)
            return carry

        lax.fori_loop(0, _NBATCH, batch, 0)

    plsc.subcore_barrier()

    @pl.when(s < NS - 1)
    def _():
        pltpu.sync_copy(tbl.at[pl.ds(s * _RPT, _RPT), :],
                        out_tbl.at[c, pl.ds(s * _RPT, _RPT), :])

    @pl.when(s == NS - 1)
    def _():
        pltpu.sync_copy(tbl.at[pl.ds(s * _RPT, _RPT_LAST), :],
                        out_tbl.at[c, pl.ds(s * _RPT, _RPT_LAST), :])


_sc_mesh = plsc.VectorSubcoreMesh(core_axis_name="c", subcore_axis_name="s",
                                  num_cores=NC, num_subcores=NS)
_sc_params = pltpu.CompilerParams(use_tc_tiling_on_sc=False)

_seg = pl.kernel(
    _seg_body,
    out_type=jax.ShapeDtypeStruct((NC, N_NODES, EDIM), jnp.float32),
    mesh=_sc_mesh,
    scratch_types=[
        pltpu.VMEM_SHARED((N_NODES, EDIM), jnp.float32),
        pltpu.VMEM((_BATCH, _IW), jnp.int32),
        pltpu.VMEM((_BATCH * _IW, EDIM), jnp.float32),
        pltpu.VMEM((_IW, EDIM), jnp.float32),
    ],
    compiler_params=_sc_params,
)

_BK = 1000


def _dense_body(tbl_ref, wn_ref, bn_ref, wp_ref, wq_ref, be_ref,
                p_ref, q_ref):
    s = tbl_ref[0]
    cnt = tbl_ref[1, :, 0:1]
    mean = s / jnp.maximum(cnt, 1.0)
    h = jnp.dot(mean, wn_ref[...], preferred_element_type=jnp.float32)
    h = jnp.maximum(h + bn_ref[...], 0.0)
    p_ref[...] = jnp.dot(h, wp_ref[...],
                         preferred_element_type=jnp.float32) + be_ref[...]
    q_ref[...] = jnp.dot(h, wq_ref[...], preferred_element_type=jnp.float32)


_dense = pl.pallas_call(
    _dense_body,
    grid=(N_NODES // _BK,),
    in_specs=[
        pl.BlockSpec((NC, _BK, EDIM), lambda i: (0, i, 0)),
        pl.BlockSpec((EDIM, NDIM), lambda i: (0, 0)),
        pl.BlockSpec((1, NDIM), lambda i: (0, 0)),
        pl.BlockSpec((NDIM, NDIM), lambda i: (0, 0)),
        pl.BlockSpec((NDIM, NDIM), lambda i: (0, 0)),
        pl.BlockSpec((1, NDIM), lambda i: (0, 0)),
    ],
    out_specs=[pl.BlockSpec((_BK, NDIM), lambda i: (i, 0)),
               pl.BlockSpec((_BK, NDIM), lambda i: (i, 0))],
    out_shape=[jax.ShapeDtypeStruct((N_NODES, NDIM), jnp.float32),
               jax.ShapeDtypeStruct((N_NODES, NDIM), jnp.float32)],
)

_NJ = 25
_PPW = _NJ * 128
_NBP = NW * _PPW


def _pair_body(pt, qt, srcp, dstp, out, isrc, idst, gp, gq, ob, sem1, sem2):
    c = lax.axis_index("c")
    s = lax.axis_index("s")
    w = c * NS + s
    pltpu.sync_copy(srcp.at[w], isrc)
    pltpu.sync_copy(dstp.at[w], idst)
    base = w * _PPW

    def issue(j, carry):
        cp1 = pltpu.async_copy(pt.at[isrc.at[j]], gp, sem1)
        cp2 = pltpu.async_copy(qt.at[idst.at[j]], gq, sem2)
        cp1.wait()
        cp2.wait()

        def addrow(i, c2):
            for k in range(NDIM // 16):
                ob[i, pl.ds(k * 16, 16)] = (gp[i, pl.ds(k * 16, 16)] +
                                            gq[i, pl.ds(k * 16, 16)])
            return c2

        lax.fori_loop(0, 128, addrow, 0)
        pltpu.sync_copy(ob, out.at[pl.ds(base + j * 128, 128), :])
        return carry

    lax.fori_loop(0, _NJ, issue, 0)


_pair = pl.kernel(
    _pair_body,
    out_type=jax.ShapeDtypeStruct((_NBP, NDIM), jnp.float32),
    mesh=_sc_mesh,
    scratch_types=[
        pltpu.VMEM((_NJ, 128), jnp.int32),
        pltpu.VMEM((_NJ, 128), jnp.int32),
        pltpu.VMEM((128, NDIM), jnp.float32),
        pltpu.VMEM((128, NDIM), jnp.float32),
        pltpu.VMEM((128, NDIM), jnp.float32),
        pltpu.SemaphoreType.DMA,
        pltpu.SemaphoreType.DMA,
    ],
    compiler_params=pltpu.CompilerParams(use_tc_tiling_on_sc=True),
)


def kernel(x, edge_index, edge_attr, edge_couples,
           W_neigh0, b_neigh0, W_edge0, b_edge0,
           W_neigh1, b_neigh1, W_edge1, b_edge1):
    del x
    dst = edge_index[1]
    dst2 = dst.reshape(_ROWS, _IW)
    zsum = jnp.zeros((_RPT, EDIM), jnp.float32)
    ones_h = jnp.ones((_IW, EDIM), jnp.float32)
    out_tbl = _seg(dst2, edge_attr, zsum, ones_h)

    wn = W_neigh1.T
    bn = b_neigh1.reshape(1, NDIM)
    wp = W_edge1[:, :NDIM].T
    wq = W_edge1[:, NDIM:].T
    be = b_edge1.reshape(1, NDIM)
    P, Q = _dense(out_tbl, wn, bn, wp, wq, be)

    def _pack_idx(v):
        v = jnp.pad(v, (0, _NBP - NB))
        return v.reshape(NW, _NJ, 128)

    e1 = _pair(P, Q, _pack_idx(edge_couples[:, 0]), _pack_idx(edge_couples[:, 1]))
    return e1[:NB]


# speedup vs baseline: 6.0083x; 1.0142x over previous
import jax
import jax.numpy as jnp
from jax import lax
from jax.experimental import pallas as pl
from jax.experimental.pallas import tpu as pltpu
from jax.experimental.pallas import tpu_sc as plsc

N_NODES = 100000
N_EDGES = 3200000
EDIM = 16
NDIM = 128
NB = 100000

NC = 2
NS = 16
NW = NC * NS

_IW = 125
_ROWS = N_EDGES // _IW
_RPT16 = _ROWS // NS
_BATCH = 8
_NBATCH = _RPT16 // _BATCH
_RPT = 6256
_RPT_LAST = N_NODES - 15 * _RPT


def _seg_body(dst2, attr, zsum, ones_h, out_tbl,
              tbl, idx2, attr_v, ones_v):
    c = lax.axis_index("c")
    s = lax.axis_index("s")
    @pl.when(s < NS - 1)
    def _():
        pltpu.sync_copy(zsum, tbl.at[pl.ds(s * _RPT, _RPT), :])

    @pl.when(s == NS - 1)
    def _():
        pltpu.sync_copy(zsum.at[pl.ds(0, _RPT_LAST), :],
                        tbl.at[pl.ds(s * _RPT, _RPT_LAST), :])

    pltpu.sync_copy(ones_h, ones_v)
    plsc.subcore_barrier()

    row0 = s * _RPT16

    @pl.when(c == 0)
    def _():
        def batch(b, carry):
            r0 = row0 + b * _BATCH
            e0 = r0 * _IW
            pltpu.sync_copy(dst2.at[pl.ds(r0, _BATCH), :], idx2)
            pltpu.sync_copy(attr.at[pl.ds(e0, _BATCH * _IW), :], attr_v)
            for j in range(_BATCH):
                pltpu.sync_copy(attr_v.at[pl.ds(j * _IW, _IW), :],
                                tbl.at[idx2.at[j]], add=True)
            return carry

        lax.fori_loop(0, _NBATCH, batch, 0)

    @pl.when(c == 1)
    def _():
        def batch(b, carry):
            r0 = row0 + b * _BATCH
            pltpu.sync_copy(dst2.at[pl.ds(r0, _BATCH), :], idx2)
            for j in range(_BATCH):
                pltpu.sync_copy(ones_v, tbl.at[idx2.at[j]], add=True)
            return carry

        lax.fori_loop(0, _NBATCH, batch, 0)

    plsc.subcore_barrier()

    @pl.when(s < NS - 1)
    def _():
        pltpu.sync_copy(tbl.at[pl.ds(s * _RPT, _RPT), :],
                        out_tbl.at[c, pl.ds(s * _RPT, _RPT), :])

    @pl.when(s == NS - 1)
    def _():
        pltpu.sync_copy(tbl.at[pl.ds(s * _RPT, _RPT_LAST), :],
                        out_tbl.at[c, pl.ds(s * _RPT, _RPT_LAST), :])


_sc_mesh = plsc.VectorSubcoreMesh(core_axis_name="c", subcore_axis_name="s",
                                  num_cores=NC, num_subcores=NS)
_sc_params = pltpu.CompilerParams(use_tc_tiling_on_sc=False)

_seg = pl.kernel(
    _seg_body,
    out_type=jax.ShapeDtypeStruct((NC, N_NODES, EDIM), jnp.float32),
    mesh=_sc_mesh,
    scratch_types=[
        pltpu.VMEM_SHARED((N_NODES, EDIM), jnp.float32),
        pltpu.VMEM((_BATCH, _IW), jnp.int32),
        pltpu.VMEM((_BATCH * _IW, EDIM), jnp.float32),
        pltpu.VMEM((_IW, EDIM), jnp.float32),
    ],
    compiler_params=_sc_params,
)

_BK = 1000


def _mean_body(tbl_ref, mean_ref):
    s = tbl_ref[0]
    cnt = tbl_ref[1, :, 0:1]
    mean_ref[...] = s / jnp.maximum(cnt, 1.0)


_mean = pl.pallas_call(
    _mean_body,
    grid=(N_NODES // _BK,),
    in_specs=[pl.BlockSpec((NC, _BK, EDIM), lambda i: (0, i, 0))],
    out_specs=pl.BlockSpec((_BK, EDIM), lambda i: (i, 0)),
    out_shape=jax.ShapeDtypeStruct((N_NODES, EDIM), jnp.float32),
)

_CPW = 3128
_NJ = 25


def _pair_body(mt, srcp, dstp, outs, outd, isrc, idst, gs, gd, sem1, sem2):
    c = lax.axis_index("c")
    s = lax.axis_index("s")
    w = c * NS + s
    pltpu.sync_copy(srcp.at[w], isrc)
    pltpu.sync_copy(dstp.at[w], idst)
    base = w * _CPW
    lastw = w == NW - 1
    nf = jnp.where(lastw, _NJ - 2, _NJ - 1)

    def issue(j, carry):
        cp1 = pltpu.async_copy(mt.at[isrc.at[j]], gs, sem1)
        cp2 = pltpu.async_copy(mt.at[idst.at[j]], gd, sem2)
        cp1.wait()
        cp2.wait()

        @pl.when(j < nf)
        def _():
            pltpu.sync_copy(gs, outs.at[pl.ds(base + j * 128, 128), :])
            pltpu.sync_copy(gd, outd.at[pl.ds(base + j * 128, 128), :])

        @pl.when((j == nf) & jnp.logical_not(lastw))
        def _():
            pltpu.sync_copy(gs.at[pl.ds(0, 56), :],
                            outs.at[pl.ds(base + j * 128, 56), :])
            pltpu.sync_copy(gd.at[pl.ds(0, 56), :],
                            outd.at[pl.ds(base + j * 128, 56), :])

        @pl.when((j == nf) & lastw)
        def _():
            pltpu.sync_copy(gs.at[pl.ds(0, 88), :],
                            outs.at[pl.ds(base + j * 128, 88), :])
            pltpu.sync_copy(gd.at[pl.ds(0, 88), :],
                            outd.at[pl.ds(base + j * 128, 88), :])

        return carry

    lax.fori_loop(0, _NJ, issue, 0)


_pair = pl.kernel(
    _pair_body,
    out_type=(jax.ShapeDtypeStruct((NB, EDIM), jnp.float32),
              jax.ShapeDtypeStruct((NB, EDIM), jnp.float32)),
    mesh=_sc_mesh,
    scratch_types=[
        pltpu.VMEM((_NJ, 128), jnp.int32),
        pltpu.VMEM((_NJ, 128), jnp.int32),
        pltpu.VMEM((128, EDIM), jnp.float32),
        pltpu.VMEM((128, EDIM), jnp.float32),
        pltpu.SemaphoreType.DMA,
        pltpu.SemaphoreType.DMA,
    ],
    compiler_params=_sc_params,
)


def _edge_body(ms_ref, md_ref, wn_ref, bn_ref, wp_ref, wq_ref, be_ref,
               e_ref):
    hs = jnp.dot(ms_ref[...], wn_ref[...], preferred_element_type=jnp.float32)
    hs = jnp.maximum(hs + bn_ref[...], 0.0)
    hd = jnp.dot(md_ref[...], wn_ref[...], preferred_element_type=jnp.float32)
    hd = jnp.maximum(hd + bn_ref[...], 0.0)
    e_ref[...] = (jnp.dot(hs, wp_ref[...], preferred_element_type=jnp.float32)
                  + jnp.dot(hd, wq_ref[...], preferred_element_type=jnp.float32)
                  + be_ref[...])


_edge = pl.pallas_call(
    _edge_body,
    grid=(NB // _BK,),
    in_specs=[
        pl.BlockSpec((_BK, EDIM), lambda i: (i, 0)),
        pl.BlockSpec((_BK, EDIM), lambda i: (i, 0)),
        pl.BlockSpec((EDIM, NDIM), lambda i: (0, 0)),
        pl.BlockSpec((1, NDIM), lambda i: (0, 0)),
        pl.BlockSpec((NDIM, NDIM), lambda i: (0, 0)),
        pl.BlockSpec((NDIM, NDIM), lambda i: (0, 0)),
        pl.BlockSpec((1, NDIM), lambda i: (0, 0)),
    ],
    out_specs=pl.BlockSpec((_BK, NDIM), lambda i: (i, 0)),
    out_shape=jax.ShapeDtypeStruct((NB, NDIM), jnp.float32),
)


def kernel(x, edge_index, edge_attr, edge_couples,
           W_neigh0, b_neigh0, W_edge0, b_edge0,
           W_neigh1, b_neigh1, W_edge1, b_edge1):
    del x
    dst = edge_index[1]
    dst2 = dst.reshape(_ROWS, _IW)
    zsum = jnp.zeros((_RPT, EDIM), jnp.float32)
    ones_h = jnp.ones((_IW, EDIM), jnp.float32)
    out_tbl = _seg(dst2, edge_attr, zsum, ones_h)
    mean = _mean(out_tbl)

    def _pack_idx(v):
        v = jnp.pad(v, (0, NW * _CPW + (_NJ * 128 - _CPW) - NB))
        pos = ((jnp.arange(NW) * _CPW)[:, None, None]
               + (jnp.arange(_NJ) * 128)[None, :, None]
               + jnp.arange(128)[None, None, :])
        return v[pos]

    ms, md = _pair(mean, _pack_idx(edge_couples[:, 0]),
                   _pack_idx(edge_couples[:, 1]))

    wn = W_neigh1.T
    bn = b_neigh1.reshape(1, NDIM)
    wp = W_edge1[:, :NDIM].T
    wq = W_edge1[:, NDIM:].T
    be = b_edge1.reshape(1, NDIM)
    return _edge(ms, md, wn, bn, wp, wq, be)


# speedup vs baseline: 6.4483x; 1.0732x over previous
import jax
import jax.numpy as jnp
from jax import lax
from jax.experimental import pallas as pl
from jax.experimental.pallas import tpu as pltpu
from jax.experimental.pallas import tpu_sc as plsc

N_NODES = 100000
N_EDGES = 3200000
EDIM = 16
NDIM = 128
NB = 100000

NC = 2
NS = 16
NW = NC * NS

_IW = 125
_ROWS = N_EDGES // _IW
_RPT16 = _ROWS // NS
_BATCH = 4
_NBATCH = _RPT16 // _BATCH
_RPT = 6256
_RPT_LAST = N_NODES - 15 * _RPT


def _seg_body(dst2, attr, zsum, ones_h, out_tbl,
              tbl, idx2, attr_v, ones_v, idx2b, attr_vb,
              semi, sema, semib, semab):
    c = lax.axis_index("c")
    s = lax.axis_index("s")
    @pl.when(s < NS - 1)
    def _():
        pltpu.sync_copy(zsum, tbl.at[pl.ds(s * _RPT, _RPT), :])

    @pl.when(s == NS - 1)
    def _():
        pltpu.sync_copy(zsum.at[pl.ds(0, _RPT_LAST), :],
                        tbl.at[pl.ds(s * _RPT, _RPT_LAST), :])

    pltpu.sync_copy(ones_h, ones_v)
    plsc.subcore_barrier()

    row0 = s * _RPT16

    @pl.when(c == 0)
    def _():
        pltpu.async_copy(dst2.at[pl.ds(row0, _BATCH), :], idx2, semi)
        pltpu.async_copy(attr.at[pl.ds(row0 * _IW, _BATCH * _IW), :],
                         attr_v, sema)

        def batch2(b2, carry):
            b = b2 * 2
            r0 = row0 + b * _BATCH
            r1 = r0 + _BATCH
            r2 = r1 + _BATCH
            pltpu.make_async_copy(dst2.at[pl.ds(r0, _BATCH), :],
                                  idx2, semi).wait()
            pltpu.make_async_copy(attr.at[pl.ds(r0 * _IW, _BATCH * _IW), :],
                                  attr_v, sema).wait()
            pltpu.async_copy(dst2.at[pl.ds(r1, _BATCH), :], idx2b, semib)
            pltpu.async_copy(attr.at[pl.ds(r1 * _IW, _BATCH * _IW), :],
                             attr_vb, semab)
            for j in range(_BATCH):
                pltpu.sync_copy(attr_v.at[pl.ds(j * _IW, _IW), :],
                                tbl.at[idx2.at[j]], add=True)
            pltpu.make_async_copy(dst2.at[pl.ds(r1, _BATCH), :],
                                  idx2b, semib).wait()
            pltpu.make_async_copy(attr.at[pl.ds(r1 * _IW, _BATCH * _IW), :],
                                  attr_vb, semab).wait()

            @pl.when(b2 < _NBATCH // 2 - 1)
            def _():
                pltpu.async_copy(dst2.at[pl.ds(r2, _BATCH), :], idx2, semi)
                pltpu.async_copy(attr.at[pl.ds(r2 * _IW, _BATCH * _IW), :],
                                 attr_v, sema)

            for j in range(_BATCH):
                pltpu.sync_copy(attr_vb.at[pl.ds(j * _IW, _IW), :],
                                tbl.at[idx2b.at[j]], add=True)
            return carry

        lax.fori_loop(0, _NBATCH // 2, batch2, 0)

    @pl.when(c == 1)
    def _():
        def batch(b, carry):
            r0 = row0 + b * _BATCH
            pltpu.sync_copy(dst2.at[pl.ds(r0, _BATCH), :], idx2)
            for j in range(_BATCH):
                pltpu.sync_copy(ones_v, tbl.at[idx2.at[j]], add=True)
            return carry

        lax.fori_loop(0, _NBATCH, batch, 0)

    plsc.subcore_barrier()

    @pl.when(s < NS - 1)
    def _():
        pltpu.sync_copy(tbl.at[pl.ds(s * _RPT, _RPT), :],
                        out_tbl.at[c, pl.ds(s * _RPT, _RPT), :])

    @pl.when(s == NS - 1)
    def _():
        pltpu.sync_copy(tbl.at[pl.ds(s * _RPT, _RPT_LAST), :],
                        out_tbl.at[c, pl.ds(s * _RPT, _RPT_LAST), :])


_sc_mesh = plsc.VectorSubcoreMesh(core_axis_name="c", subcore_axis_name="s",
                                  num_cores=NC, num_subcores=NS)
_sc_params = pltpu.CompilerParams(use_tc_tiling_on_sc=False)

_seg = pl.kernel(
    _seg_body,
    out_type=jax.ShapeDtypeStruct((NC, N_NODES, EDIM), jnp.float32),
    mesh=_sc_mesh,
    scratch_types=[
        pltpu.VMEM_SHARED((N_NODES, EDIM), jnp.float32),
        pltpu.VMEM((_BATCH, _IW), jnp.int32),
        pltpu.VMEM((_BATCH * _IW, EDIM), jnp.float32),
        pltpu.VMEM((_IW, EDIM), jnp.float32),
        pltpu.VMEM((_BATCH, _IW), jnp.int32),
        pltpu.VMEM((_BATCH * _IW, EDIM), jnp.float32),
        pltpu.SemaphoreType.DMA,
        pltpu.SemaphoreType.DMA,
        pltpu.SemaphoreType.DMA,
        pltpu.SemaphoreType.DMA,
    ],
    compiler_params=_sc_params,
)

_BK = 1000


def _mean_body(tbl_ref, mean_ref):
    s = tbl_ref[0]
    cnt = tbl_ref[1, :, 0:1]
    mean_ref[...] = s / jnp.maximum(cnt, 1.0)


_mean = pl.pallas_call(
    _mean_body,
    grid=(N_NODES // _BK,),
    in_specs=[pl.BlockSpec((NC, _BK, EDIM), lambda i: (0, i, 0))],
    out_specs=pl.BlockSpec((_BK, EDIM), lambda i: (i, 0)),
    out_shape=jax.ShapeDtypeStruct((N_NODES, EDIM), jnp.float32),
)

_CPW = 3128
_NJ = 25


def _pair_body(mt, srcp, dstp, outs, outd, isrc, idst, gs, gd, sem1, sem2):
    c = lax.axis_index("c")
    s = lax.axis_index("s")
    w = c * NS + s
    pltpu.sync_copy(srcp.at[w], isrc)
    pltpu.sync_copy(dstp.at[w], idst)
    base = w * _CPW
    lastw = w == NW - 1
    nf = jnp.where(lastw, _NJ - 2, _NJ - 1)

    def issue(j, carry):
        cp1 = pltpu.async_copy(mt.at[isrc.at[j]], gs, sem1)
        cp2 = pltpu.async_copy(mt.at[idst.at[j]], gd, sem2)
        cp1.wait()
        cp2.wait()

        @pl.when(j < nf)
        def _():
            pltpu.sync_copy(gs, outs.at[pl.ds(base + j * 128, 128), :])
            pltpu.sync_copy(gd, outd.at[pl.ds(base + j * 128, 128), :])

        @pl.when((j == nf) & jnp.logical_not(lastw))
        def _():
            pltpu.sync_copy(gs.at[pl.ds(0, 56), :],
                            outs.at[pl.ds(base + j * 128, 56), :])
            pltpu.sync_copy(gd.at[pl.ds(0, 56), :],
                            outd.at[pl.ds(base + j * 128, 56), :])

        @pl.when((j == nf) & lastw)
        def _():
            pltpu.sync_copy(gs.at[pl.ds(0, 88), :],
                            outs.at[pl.ds(base + j * 128, 88), :])
            pltpu.sync_copy(gd.at[pl.ds(0, 88), :],
                            outd.at[pl.ds(base + j * 128, 88), :])

        return carry

    lax.fori_loop(0, _NJ, issue, 0)


_pair = pl.kernel(
    _pair_body,
    out_type=(jax.ShapeDtypeStruct((NB, EDIM), jnp.float32),
              jax.ShapeDtypeStruct((NB, EDIM), jnp.float32)),
    mesh=_sc_mesh,
    scratch_types=[
        pltpu.VMEM((_NJ, 128), jnp.int32),
        pltpu.VMEM((_NJ, 128), jnp.int32),
        pltpu.VMEM((128, EDIM), jnp.float32),
        pltpu.VMEM((128, EDIM), jnp.float32),
        pltpu.SemaphoreType.DMA,
        pltpu.SemaphoreType.DMA,
    ],
    compiler_params=_sc_params,
)


def _edge_body(ms_ref, md_ref, wn_ref, bn_ref, wp_ref, wq_ref, be_ref,
               e_ref):
    hs = jnp.dot(ms_ref[...], wn_ref[...], preferred_element_type=jnp.float32)
    hs = jnp.maximum(hs + bn_ref[...], 0.0)
    hd = jnp.dot(md_ref[...], wn_ref[...], preferred_element_type=jnp.float32)
    hd = jnp.maximum(hd + bn_ref[...], 0.0)
    e_ref[...] = (jnp.dot(hs, wp_ref[...], preferred_element_type=jnp.float32)
                  + jnp.dot(hd, wq_ref[...], preferred_element_type=jnp.float32)
                  + be_ref[...])


_edge = pl.pallas_call(
    _edge_body,
    grid=(NB // _BK,),
    in_specs=[
        pl.BlockSpec((_BK, EDIM), lambda i: (i, 0)),
        pl.BlockSpec((_BK, EDIM), lambda i: (i, 0)),
        pl.BlockSpec((EDIM, NDIM), lambda i: (0, 0)),
        pl.BlockSpec((1, NDIM), lambda i: (0, 0)),
        pl.BlockSpec((NDIM, NDIM), lambda i: (0, 0)),
        pl.BlockSpec((NDIM, NDIM), lambda i: (0, 0)),
        pl.BlockSpec((1, NDIM), lambda i: (0, 0)),
    ],
    out_specs=pl.BlockSpec((_BK, NDIM), lambda i: (i, 0)),
    out_shape=jax.ShapeDtypeStruct((NB, NDIM), jnp.float32),
)


def kernel(x, edge_index, edge_attr, edge_couples,
           W_neigh0, b_neigh0, W_edge0, b_edge0,
           W_neigh1, b_neigh1, W_edge1, b_edge1):
    del x
    dst = edge_index[1]
    dst2 = dst.reshape(_ROWS, _IW)
    zsum = jnp.zeros((_RPT, EDIM), jnp.float32)
    ones_h = jnp.ones((_IW, EDIM), jnp.float32)
    out_tbl = _seg(dst2, edge_attr, zsum, ones_h)
    mean = _mean(out_tbl)

    def _pack_idx(v):
        v = jnp.pad(v, (0, NW * _CPW + (_NJ * 128 - _CPW) - NB))
        pos = ((jnp.arange(NW) * _CPW)[:, None, None]
               + (jnp.arange(_NJ) * 128)[None, :, None]
               + jnp.arange(128)[None, None, :])
        return v[pos]

    ms, md = _pair(mean, _pack_idx(edge_couples[:, 0]),
                   _pack_idx(edge_couples[:, 1]))

    wn = W_neigh1.T
    bn = b_neigh1.reshape(1, NDIM)
    wp = W_edge1[:, :NDIM].T
    wq = W_edge1[:, NDIM:].T
    be = b_edge1.reshape(1, NDIM)
    return _edge(ms, md, wn, bn, wp, wq, be)
